# baseline (speedup 1.0000x reference)
import functools
import jax
import jax.numpy as jnp
from jax import lax
from jax.experimental import pallas as pl

N = 50000
E = 800000
G = 500
NPG = 100
K1 = 80
K2 = 64

GB = 25
GBP = 5
EPS = 1e-5
SELU_A = 1.6732632423543772
SELU_S = 1.0507009873554805


def _bmm(a, b):
    return lax.dot_general(a, b, (((2,), (1,)), ((0,), (0,))),
                           precision=lax.Precision.HIGHEST,
                           preferred_element_type=jnp.float32)


def _bf16mm(a, b):
    return jnp.dot(a.astype(jnp.bfloat16), b.astype(jnp.bfloat16),
                   preferred_element_type=jnp.float32)


def _b16(t):
    return t.astype(jnp.bfloat16).astype(jnp.float32)


def _full(shape):
    rank = len(shape)
    return pl.BlockSpec(shape, lambda i, _r=rank: (0,) * _r)


def _gblk(shape, gb):
    rank = len(shape)
    return pl.BlockSpec((gb,) + shape[1:],
                        lambda i, _r=rank: (i,) + (0,) * (_r - 1))


def _gout3(d, gb):
    return pl.BlockSpec((1, gb, d), lambda i: (i, 0, 0))


def _bn_apply(z, stats, g, b, nrows):
    mu = stats[0, :] / nrows
    var = stats[1, :] / nrows - mu * mu
    inv = g * lax.rsqrt(var + EPS)
    return (z - mu) * inv + b


def _acc_stats(ref, z2d):
    @pl.when(pl.program_id(0) == 0)
    def _():
        ref[...] = jnp.zeros_like(ref)
    ref[0, :] += jnp.sum(z2d, axis=0)
    ref[1, :] += jnp.sum(z2d * z2d, axis=0)


def _selu(x):
    return SELU_S * jnp.where(x > 0, x, SELU_A * (jnp.exp(x) - 1.0))


def _k1(gb, x_r, agg_r, w_r, b_r, z_r, st_r):
    z = x_r[...] + agg_r[...]
    z2 = _bf16mm(z.reshape(gb * NPG, 15), w_r[...]) + b_r[...]
    z_r[...] = z2.reshape(gb, NPG, 20)
    _acc_stats(st_r, z2)


def _k_bn_mm(gb, nrows, din, dout, npg, z_r, st_r, g_r, be_r, w_r, b_r, o_r, st2_r):
    z = z_r[...].reshape(gb * npg, din)
    h = jnp.maximum(_bn_apply(z, st_r[...], g_r[0, :], be_r[0, :], nrows), 0.0)
    z2 = _bf16mm(h, w_r[...]) + b_r[...]
    o_r[...] = z2.reshape(gb, npg, dout)
    _acc_stats(st2_r, z2)


def _k_bn_relu(gb, nrows, din, npg, z_r, st_r, g_r, be_r, o_r, st2_r):
    z = z_r[...].reshape(gb * npg, din)
    h = jnp.maximum(_bn_apply(z, st_r[...], g_r[0, :], be_r[0, :], nrows), 0.0)
    o_r[...] = h.reshape(gb, npg, din)
    _acc_stats(st2_r, h)


def _k_selu_bn(gb, nrows, din, npg, z_r, st_r, g_r, be_r, o_r):
    z = z_r[...].reshape(gb * npg, din)
    o_r[...] = _selu(_bn_apply(z, st_r[...], g_r[0, :], be_r[0, :], nrows)
                     ).reshape(gb, npg, din)


def _rank(s, gb, npg):
    si = s[:, :, None]
    sj = s[:, None, :]
    ii = lax.broadcasted_iota(jnp.int32, (gb, npg, npg), 1)
    jj = lax.broadcasted_iota(jnp.int32, (gb, npg, npg), 2)
    gt = (sj > si) | ((sj == si) & (jj < ii))
    return jnp.sum(gt.astype(jnp.int32), axis=2)


def _k_pool(gb, npg, k, x_r, aggs_r, wrel_r, brel_r, wroot_r,
            hp_r, r_r, rank_r):
    xb = x_r[...]
    aggs = aggs_r[...]
    score = (jnp.sum(_b16(aggs) * _b16(wrel_r[...]), axis=2) + brel_r[0, 0]
             + jnp.sum(_b16(xb) * _b16(wroot_r[...]), axis=2))
    rank = _rank(score, gb, npg)
    p = (rank[:, None, :] ==
         lax.broadcasted_iota(jnp.int32, (gb, k, npg), 1)).astype(jnp.float32)
    hp = _bmm(p, xb * jnp.tanh(score)[:, :, None])
    hp_r[...] = hp
    r_r[...] = jnp.sum(hp, axis=1)[None]
    rank_r[...] = rank[None]


def _k8(r1_r, r2_r, mol_r, w1a_r, w1b_r, w1c_r, b1_r, g1_r, be1_r,
        w2_r, b2_r, g2_r, be2_r, w3_r, b3_r, o_r):
    z = (_bf16mm(r1_r[...], w1a_r[...]) + _bf16mm(r2_r[...], w1b_r[...])
         + _bf16mm(mol_r[...], w1c_r[...]) + b1_r[...])
    mu = jnp.mean(z, axis=0)
    var = jnp.mean(z * z, axis=0) - mu * mu
    h = jnp.maximum(g1_r[0, :] * (z - mu) * lax.rsqrt(var + EPS) + be1_r[0, :], 0.0)
    z2 = _bf16mm(h, w2_r[...]) + b2_r[...]
    mu2 = jnp.mean(z2, axis=0)
    var2 = jnp.mean(z2 * z2, axis=0) - mu2 * mu2
    h2 = jnp.maximum(g2_r[0, :] * (z2 - mu2) * lax.rsqrt(var2 + EPS) + be2_r[0, :], 0.0)
    o_r[...] = _bf16mm(h2, w3_r[...]) + b3_r[...]


def _call(body, gb, ins, outs):
    in_specs = [spec for spec, _ in ins]
    out_specs = tuple(spec for spec, _, _dt in outs)
    out_shapes = tuple(jax.ShapeDtypeStruct(shp, dt) for _, shp, dt in outs)
    return pl.pallas_call(
        functools.partial(body, gb), grid=(G // gb,), in_specs=in_specs,
        out_specs=out_specs, out_shape=out_shapes,
    )(*[a for _, a in ins])


def kernel(x, edge_index, mol_features, batch, params):
    p = params
    f32 = jnp.float32
    src = edge_index[0].astype(jnp.int32)
    dst = edge_index[1].astype(jnp.int32)
    ew0 = jnp.ones((E,), f32)
    row = lambda v: v.reshape(1, -1)

    agg1 = jax.ops.segment_sum(x[src] * ew0[:, None], dst, num_segments=N)

    z1_raw, st1 = _call(
        _k1, GB,
        [(_gblk((G, NPG, 15), GB), x.reshape(G, NPG, 15)),
         (_gblk((G, NPG, 15), GB), agg1.reshape(G, NPG, 15)),
         (_full((15, 20)), p['c1_W1']), (_full((1, 20)), row(p['c1_b1']))],
        [(_gblk((G, NPG, 20), GB), (G, NPG, 20), f32),
         (_full((2, 20)), (2, 20), f32)])

    z2_raw, st2 = _call(
        lambda gb, *a: _k_bn_mm(gb, N, 20, 20, NPG, *a), GB,
        [(_gblk((G, NPG, 20), GB), z1_raw), (_full((2, 20)), st1),
         (_full((1, 20)), row(p['c1_g1'])), (_full((1, 20)), row(p['c1_be1'])),
         (_full((20, 20)), p['c1_W2']), (_full((1, 20)), row(p['c1_b2']))],
        [(_gblk((G, NPG, 20), GB), (G, NPG, 20), f32),
         (_full((2, 20)), (2, 20), f32)])

    x1_raw, st3 = _call(
        lambda gb, *a: _k_bn_relu(gb, N, 20, NPG, *a), GB,
        [(_gblk((G, NPG, 20), GB), z2_raw), (_full((2, 20)), st2),
         (_full((1, 20)), row(p['c1_g2'])), (_full((1, 20)), row(p['c1_be2']))],
        [(_gblk((G, NPG, 20), GB), (G, NPG, 20), f32),
         (_full((2, 20)), (2, 20), f32)])

    x1_b, = _call(
        lambda gb, *a: _k_selu_bn(gb, N, 20, NPG, *a), GB,
        [(_gblk((G, NPG, 20), GB), x1_raw), (_full((2, 20)), st3),
         (_full((1, 20)), row(p['bn1_g'])), (_full((1, 20)), row(p['bn1_b']))],
        [(_gblk((G, NPG, 20), GB), (G, NPG, 20), f32)])
    x1 = x1_b.reshape(N, 20)

    aggs1 = jax.ops.segment_sum(x1[src] * ew0[:, None], dst, num_segments=N)

    hp, r1, rank1 = _call(
        lambda gb, *a: _k_pool(gb, NPG, K1, *a), GBP,
        [(_gblk((G, NPG, 20), GBP), x1_b),
         (_gblk((G, NPG, 20), GBP), aggs1.reshape(G, NPG, 20)),
         (_full((1, 1, 20)), p['p1_Wrel'].reshape(1, 1, 20)),
         (_full((1, 1)), p['p1_brel'].reshape(1, 1)),
         (_full((1, 1, 20)), p['p1_Wroot'].reshape(1, 1, 20))],
        [(_gblk((G, K1, 20), GBP), (G, K1, 20), f32),
         (_gout3(20, GBP), (G // GBP, GBP, 20), f32),
         (_gout3(NPG, GBP), (G // GBP, GBP, NPG), jnp.int32)])

    rank1 = rank1.reshape(N)
    newid = jnp.where(rank1 < K1, (jnp.arange(N, dtype=jnp.int32) // NPG) * K1
                      + rank1, -1)
    keep = (newid[src] >= 0) & (newid[dst] >= 0)
    nsrc = jnp.where(keep, newid[src], 0)
    ndst = jnp.where(keep, newid[dst], 0)
    ew1 = ew0 * keep.astype(f32)

    x1p = hp.reshape(G * K1, 20)
    agg2 = jax.ops.segment_sum(x1p[nsrc] * ew1[:, None], ndst,
                               num_segments=G * K1)

    def _k_add_mm(gb, h_r, a_r, w_r, b_r, o_r, st_r):
        z = (h_r[...] + a_r[...]).reshape(gb * K1, 20)
        z2 = _bf16mm(z, w_r[...]) + b_r[...]
        o_r[...] = z2.reshape(gb, K1, 155)
        _acc_stats(st_r, z2)

    z3_raw, st4 = _call(
        _k_add_mm, GB,
        [(_gblk((G, K1, 20), GB), hp),
         (_gblk((G, K1, 20), GB), agg2.reshape(G, K1, 20)),
         (_full((20, 155)), p['c2_W1']), (_full((1, 155)), row(p['c2_b1']))],
        [(_gblk((G, K1, 155), GB), (G, K1, 155), f32),
         (_full((2, 155)), (2, 155), f32)])

    z4_raw, st5 = _call(
        lambda gb, *a: _k_bn_mm(gb, G * K1, 155, 155, K1, *a), GB,
        [(_gblk((G, K1, 155), GB), z3_raw), (_full((2, 155)), st4),
         (_full((1, 155)), row(p['c2_g1'])), (_full((1, 155)), row(p['c2_be1'])),
         (_full((155, 155)), p['c2_W2']), (_full((1, 155)), row(p['c2_b2']))],
        [(_gblk((G, K1, 155), GB), (G, K1, 155), f32),
         (_full((2, 155)), (2, 155), f32)])

    x2_raw, st6 = _call(
        lambda gb, *a: _k_bn_relu(gb, G * K1, 155, K1, *a), GB,
        [(_gblk((G, K1, 155), GB), z4_raw), (_full((2, 155)), st5),
         (_full((1, 155)), row(p['c2_g2'])), (_full((1, 155)), row(p['c2_be2']))],
        [(_gblk((G, K1, 155), GB), (G, K1, 155), f32),
         (_full((2, 155)), (2, 155), f32)])

    x2_b, = _call(
        lambda gb, *a: _k_selu_bn(gb, G * K1, 155, K1, *a), GB,
        [(_gblk((G, K1, 155), GB), x2_raw), (_full((2, 155)), st6),
         (_full((1, 155)), row(p['bn2_g'])), (_full((1, 155)), row(p['bn2_b']))],
        [(_gblk((G, K1, 155), GB), (G, K1, 155), f32)])
    x2 = x2_b.reshape(G * K1, 155)

    aggs2 = jax.ops.segment_sum(x2[nsrc] * ew1[:, None], ndst,
                                num_segments=G * K1)

    hp2, r2, _rank2o = _call(
        lambda gb, *a: _k_pool(gb, K1, K2, *a), GBP,
        [(_gblk((G, K1, 155), GBP), x2_b),
         (_gblk((G, K1, 155), GBP), aggs2.reshape(G, K1, 155)),
         (_full((1, 1, 155)), p['p2_Wrel'].reshape(1, 1, 155)),
         (_full((1, 1)), p['p2_brel'].reshape(1, 1)),
         (_full((1, 1, 155)), p['p2_Wroot'].reshape(1, 1, 155))],
        [(_gblk((G, K2, 155), GBP), (G, K2, 155), f32),
         (_gout3(155, GBP), (G // GBP, GBP, 155), f32),
         (_gout3(K1, GBP), (G // GBP, GBP, K1), jnp.int32)])

    out = pl.pallas_call(
        _k8,
        out_shape=jax.ShapeDtypeStruct((G, 138), f32),
    )(r1.reshape(G, 20), r2.reshape(G, 155), mol_features,
      p['m_W1'][:20], p['m_W1'][20:175], p['m_W1'][175:], row(p['m_b1']),
      row(p['m_g1']), row(p['m_be1']),
      p['m_W2'], row(p['m_b2']), row(p['m_g2']), row(p['m_be2']),
      p['m_W3'], row(p['m_b3']))
    return out


# speedup vs baseline: 1.0070x; 1.0070x over previous
import functools
import jax
import jax.numpy as jnp
from jax import lax
from jax.experimental import pallas as pl

N = 50000
E = 800000
G = 500
NPG = 100
K1 = 80
K2 = 64

GB = 50
GBP = 10
EPS = 1e-5
SELU_A = 1.6732632423543772
SELU_S = 1.0507009873554805


def _bmm(a, b):
    return lax.dot_general(a, b, (((2,), (1,)), ((0,), (0,))),
                           precision=lax.Precision.HIGHEST,
                           preferred_element_type=jnp.float32)


def _bf16mm(a, b):
    return jnp.dot(a.astype(jnp.bfloat16), b.astype(jnp.bfloat16),
                   preferred_element_type=jnp.float32)


def _b16(t):
    return t.astype(jnp.bfloat16).astype(jnp.float32)


def _full(shape):
    rank = len(shape)
    return pl.BlockSpec(shape, lambda i, _r=rank: (0,) * _r)


def _gblk(shape, gb):
    rank = len(shape)
    return pl.BlockSpec((gb,) + shape[1:],
                        lambda i, _r=rank: (i,) + (0,) * (_r - 1))


def _gout3(d, gb):
    return pl.BlockSpec((1, gb, d), lambda i: (i, 0, 0))


def _bn_apply(z, stats, g, b, nrows):
    mu = stats[0, :] / nrows
    var = stats[1, :] / nrows - mu * mu
    inv = g * lax.rsqrt(var + EPS)
    return (z - mu) * inv + b


def _acc_stats(ref, z2d):
    @pl.when(pl.program_id(0) == 0)
    def _():
        ref[...] = jnp.zeros_like(ref)
    ref[0, :] += jnp.sum(z2d, axis=0)
    ref[1, :] += jnp.sum(z2d * z2d, axis=0)


def _selu(x):
    return SELU_S * jnp.where(x > 0, x, SELU_A * (jnp.exp(x) - 1.0))


def _k1(gb, x_r, agg_r, w_r, b_r, z_r, st_r):
    z = x_r[...] + agg_r[...]
    z2 = _bf16mm(z.reshape(gb * NPG, 15), w_r[...]) + b_r[...]
    z_r[...] = z2.reshape(gb, NPG, 20)
    _acc_stats(st_r, z2)


def _k_bn_mm(gb, nrows, din, dout, npg, z_r, st_r, g_r, be_r, w_r, b_r, o_r, st2_r):
    z = z_r[...].reshape(gb * npg, din)
    h = jnp.maximum(_bn_apply(z, st_r[...], g_r[0, :], be_r[0, :], nrows), 0.0)
    z2 = _bf16mm(h, w_r[...]) + b_r[...]
    o_r[...] = z2.reshape(gb, npg, dout)
    _acc_stats(st2_r, z2)


def _k_bn_relu(gb, nrows, din, npg, z_r, st_r, g_r, be_r, o_r, st2_r):
    z = z_r[...].reshape(gb * npg, din)
    h = jnp.maximum(_bn_apply(z, st_r[...], g_r[0, :], be_r[0, :], nrows), 0.0)
    o_r[...] = h.reshape(gb, npg, din)
    _acc_stats(st2_r, h)


def _k_selu_bn(gb, nrows, din, npg, z_r, st_r, g_r, be_r, o_r):
    z = z_r[...].reshape(gb * npg, din)
    o_r[...] = _selu(_bn_apply(z, st_r[...], g_r[0, :], be_r[0, :], nrows)
                     ).reshape(gb, npg, din)


def _rank(s, gb, npg):
    si = s[:, :, None]
    sj = s[:, None, :]
    ii = lax.broadcasted_iota(jnp.int32, (gb, npg, npg), 1)
    jj = lax.broadcasted_iota(jnp.int32, (gb, npg, npg), 2)
    gt = (sj > si) | ((sj == si) & (jj < ii))
    return jnp.sum(gt.astype(jnp.int32), axis=2)


def _k_pool(gb, npg, k, x_r, aggs_r, wrel_r, brel_r, wroot_r,
            hp_r, r_r, rank_r):
    xb = x_r[...]
    aggs = aggs_r[...]
    score = (jnp.sum(_b16(aggs) * _b16(wrel_r[...]), axis=2) + brel_r[0, 0]
             + jnp.sum(_b16(xb) * _b16(wroot_r[...]), axis=2))
    rank = _rank(score, gb, npg)
    p = (rank[:, None, :] ==
         lax.broadcasted_iota(jnp.int32, (gb, k, npg), 1)).astype(jnp.float32)
    hp = _bmm(p, xb * jnp.tanh(score)[:, :, None])
    hp_r[...] = hp
    r_r[...] = jnp.sum(hp, axis=1)[None]
    rank_r[...] = rank[None]


def _k8(r1_r, r2_r, mol_r, w1a_r, w1b_r, w1c_r, b1_r, g1_r, be1_r,
        w2_r, b2_r, g2_r, be2_r, w3_r, b3_r, o_r):
    z = (_bf16mm(r1_r[...], w1a_r[...]) + _bf16mm(r2_r[...], w1b_r[...])
         + _bf16mm(mol_r[...], w1c_r[...]) + b1_r[...])
    mu = jnp.mean(z, axis=0)
    var = jnp.mean(z * z, axis=0) - mu * mu
    h = jnp.maximum(g1_r[0, :] * (z - mu) * lax.rsqrt(var + EPS) + be1_r[0, :], 0.0)
    z2 = _bf16mm(h, w2_r[...]) + b2_r[...]
    mu2 = jnp.mean(z2, axis=0)
    var2 = jnp.mean(z2 * z2, axis=0) - mu2 * mu2
    h2 = jnp.maximum(g2_r[0, :] * (z2 - mu2) * lax.rsqrt(var2 + EPS) + be2_r[0, :], 0.0)
    o_r[...] = _bf16mm(h2, w3_r[...]) + b3_r[...]


def _call(body, gb, ins, outs):
    in_specs = [spec for spec, _ in ins]
    out_specs = tuple(spec for spec, _, _dt in outs)
    out_shapes = tuple(jax.ShapeDtypeStruct(shp, dt) for _, shp, dt in outs)
    return pl.pallas_call(
        functools.partial(body, gb), grid=(G // gb,), in_specs=in_specs,
        out_specs=out_specs, out_shape=out_shapes,
    )(*[a for _, a in ins])


def kernel(x, edge_index, mol_features, batch, params):
    p = params
    f32 = jnp.float32
    src = edge_index[0].astype(jnp.int32)
    dst = edge_index[1].astype(jnp.int32)
    ew0 = jnp.ones((E,), f32)
    row = lambda v: v.reshape(1, -1)

    agg1 = jax.ops.segment_sum(x[src] * ew0[:, None], dst, num_segments=N)

    z1_raw, st1 = _call(
        _k1, GB,
        [(_gblk((G, NPG, 15), GB), x.reshape(G, NPG, 15)),
         (_gblk((G, NPG, 15), GB), agg1.reshape(G, NPG, 15)),
         (_full((15, 20)), p['c1_W1']), (_full((1, 20)), row(p['c1_b1']))],
        [(_gblk((G, NPG, 20), GB), (G, NPG, 20), f32),
         (_full((2, 20)), (2, 20), f32)])

    z2_raw, st2 = _call(
        lambda gb, *a: _k_bn_mm(gb, N, 20, 20, NPG, *a), GB,
        [(_gblk((G, NPG, 20), GB), z1_raw), (_full((2, 20)), st1),
         (_full((1, 20)), row(p['c1_g1'])), (_full((1, 20)), row(p['c1_be1'])),
         (_full((20, 20)), p['c1_W2']), (_full((1, 20)), row(p['c1_b2']))],
        [(_gblk((G, NPG, 20), GB), (G, NPG, 20), f32),
         (_full((2, 20)), (2, 20), f32)])

    x1_raw, st3 = _call(
        lambda gb, *a: _k_bn_relu(gb, N, 20, NPG, *a), GB,
        [(_gblk((G, NPG, 20), GB), z2_raw), (_full((2, 20)), st2),
         (_full((1, 20)), row(p['c1_g2'])), (_full((1, 20)), row(p['c1_be2']))],
        [(_gblk((G, NPG, 20), GB), (G, NPG, 20), f32),
         (_full((2, 20)), (2, 20), f32)])

    x1_b, = _call(
        lambda gb, *a: _k_selu_bn(gb, N, 20, NPG, *a), GB,
        [(_gblk((G, NPG, 20), GB), x1_raw), (_full((2, 20)), st3),
         (_full((1, 20)), row(p['bn1_g'])), (_full((1, 20)), row(p['bn1_b']))],
        [(_gblk((G, NPG, 20), GB), (G, NPG, 20), f32)])
    x1 = x1_b.reshape(N, 20)

    aggs1 = jax.ops.segment_sum(x1[src] * ew0[:, None], dst, num_segments=N)

    hp, r1, rank1 = _call(
        lambda gb, *a: _k_pool(gb, NPG, K1, *a), GBP,
        [(_gblk((G, NPG, 20), GBP), x1_b),
         (_gblk((G, NPG, 20), GBP), aggs1.reshape(G, NPG, 20)),
         (_full((1, 1, 20)), p['p1_Wrel'].reshape(1, 1, 20)),
         (_full((1, 1)), p['p1_brel'].reshape(1, 1)),
         (_full((1, 1, 20)), p['p1_Wroot'].reshape(1, 1, 20))],
        [(_gblk((G, K1, 20), GBP), (G, K1, 20), f32),
         (_gout3(20, GBP), (G // GBP, GBP, 20), f32),
         (_gout3(NPG, GBP), (G // GBP, GBP, NPG), jnp.int32)])

    rank1 = rank1.reshape(N)
    newid = jnp.where(rank1 < K1, (jnp.arange(N, dtype=jnp.int32) // NPG) * K1
                      + rank1, -1)
    keep = (newid[src] >= 0) & (newid[dst] >= 0)
    nsrc = jnp.where(keep, newid[src], 0)
    ndst = jnp.where(keep, newid[dst], 0)
    ew1 = ew0 * keep.astype(f32)

    x1p = hp.reshape(G * K1, 20)
    agg2 = jax.ops.segment_sum(x1p[nsrc] * ew1[:, None], ndst,
                               num_segments=G * K1)

    def _k_add_mm(gb, h_r, a_r, w_r, b_r, o_r, st_r):
        z = (h_r[...] + a_r[...]).reshape(gb * K1, 20)
        z2 = _bf16mm(z, w_r[...]) + b_r[...]
        o_r[...] = z2.reshape(gb, K1, 155)
        _acc_stats(st_r, z2)

    z3_raw, st4 = _call(
        _k_add_mm, GB,
        [(_gblk((G, K1, 20), GB), hp),
         (_gblk((G, K1, 20), GB), agg2.reshape(G, K1, 20)),
         (_full((20, 155)), p['c2_W1']), (_full((1, 155)), row(p['c2_b1']))],
        [(_gblk((G, K1, 155), GB), (G, K1, 155), f32),
         (_full((2, 155)), (2, 155), f32)])

    z4_raw, st5 = _call(
        lambda gb, *a: _k_bn_mm(gb, G * K1, 155, 155, K1, *a), GB,
        [(_gblk((G, K1, 155), GB), z3_raw), (_full((2, 155)), st4),
         (_full((1, 155)), row(p['c2_g1'])), (_full((1, 155)), row(p['c2_be1'])),
         (_full((155, 155)), p['c2_W2']), (_full((1, 155)), row(p['c2_b2']))],
        [(_gblk((G, K1, 155), GB), (G, K1, 155), f32),
         (_full((2, 155)), (2, 155), f32)])

    x2_raw, st6 = _call(
        lambda gb, *a: _k_bn_relu(gb, G * K1, 155, K1, *a), GB,
        [(_gblk((G, K1, 155), GB), z4_raw), (_full((2, 155)), st5),
         (_full((1, 155)), row(p['c2_g2'])), (_full((1, 155)), row(p['c2_be2']))],
        [(_gblk((G, K1, 155), GB), (G, K1, 155), f32),
         (_full((2, 155)), (2, 155), f32)])

    x2_b, = _call(
        lambda gb, *a: _k_selu_bn(gb, G * K1, 155, K1, *a), GB,
        [(_gblk((G, K1, 155), GB), x2_raw), (_full((2, 155)), st6),
         (_full((1, 155)), row(p['bn2_g'])), (_full((1, 155)), row(p['bn2_b']))],
        [(_gblk((G, K1, 155), GB), (G, K1, 155), f32)])
    x2 = x2_b.reshape(G * K1, 155)

    aggs2 = jax.ops.segment_sum(x2[nsrc] * ew1[:, None], ndst,
                                num_segments=G * K1)

    hp2, r2, _rank2o = _call(
        lambda gb, *a: _k_pool(gb, K1, K2, *a), GBP,
        [(_gblk((G, K1, 155), GBP), x2_b),
         (_gblk((G, K1, 155), GBP), aggs2.reshape(G, K1, 155)),
         (_full((1, 1, 155)), p['p2_Wrel'].reshape(1, 1, 155)),
         (_full((1, 1)), p['p2_brel'].reshape(1, 1)),
         (_full((1, 1, 155)), p['p2_Wroot'].reshape(1, 1, 155))],
        [(_gblk((G, K2, 155), GBP), (G, K2, 155), f32),
         (_gout3(155, GBP), (G // GBP, GBP, 155), f32),
         (_gout3(K1, GBP), (G // GBP, GBP, K1), jnp.int32)])

    out = pl.pallas_call(
        _k8,
        out_shape=jax.ShapeDtypeStruct((G, 138), f32),
    )(r1.reshape(G, 20), r2.reshape(G, 155), mol_features,
      p['m_W1'][:20], p['m_W1'][20:175], p['m_W1'][175:], row(p['m_b1']),
      row(p['m_g1']), row(p['m_be1']),
      p['m_W2'], row(p['m_b2']), row(p['m_g2']), row(p['m_be2']),
      p['m_W3'], row(p['m_b3']))
    return out
